# baseline (speedup 1.0000x reference)
import jax
import jax.numpy as jnp
from jax.experimental import pallas as pl

NUM_EDGES = 36
NUM_NODES = 12


def _gnn_block(x_ref, prior_ref, w1_ref, b1_ref, w2_ref, w3_ref, b3_ref,
               node_ref, attn_ref):
    bb = x_ref.shape[0]
    x2 = jnp.concatenate([x_ref[:, e, :] for e in range(NUM_EDGES)], axis=0)

    h = jnp.dot(x2.astype(jnp.bfloat16), w1_ref[...].astype(jnp.bfloat16),
                preferred_element_type=jnp.float32)
    h = jnp.maximum(h + b1_ref[...], 0.0)
    lcol = jnp.sum(h * w2_ref[...], axis=1, keepdims=True)

    logits = jnp.concatenate(
        [lcol[e * bb:(e + 1) * bb] for e in range(NUM_EDGES)], axis=1)
    logits = logits + prior_ref[...]
    m = jnp.max(logits, axis=1, keepdims=True)
    p = jnp.exp(logits - m)
    attn = p / jnp.sum(p, axis=1, keepdims=True)
    attn_ref[...] = attn

    acol = jnp.concatenate(
        [attn[:, e:e + 1] for e in range(NUM_EDGES)], axis=0)
    wt = x2 * acol

    nodes = []
    for u in range(6):
        acc = wt[(6 * u) * bb:(6 * u + 1) * bb]
        for k in range(1, 6):
            acc = acc + wt[(6 * u + k) * bb:(6 * u + k + 1) * bb]
        nodes.append(acc)
    for v in range(6):
        acc = wt[v * bb:(v + 1) * bb]
        for k in range(1, 6):
            acc = acc + wt[(6 * k + v) * bb:(6 * k + v + 1) * bb]
        nodes.append(acc)
    nstack = jnp.concatenate(nodes, axis=0)

    y = jnp.dot(nstack.astype(jnp.bfloat16), w3_ref[...].astype(jnp.bfloat16),
                preferred_element_type=jnp.float32)
    y = jnp.maximum(y + b3_ref[...], 0.0)
    for n in range(NUM_NODES):
        node_ref[:, n, :] = y[n * bb:(n + 1) * bb]


def kernel(edge_feats, prior_w, W1, b1, W2, b2, W3, b3):
    B, E, D = edge_feats.shape
    hid = W1.shape[1]
    bb = 64
    while B % bb:
        bb //= 2
    grid = (B // bb,)

    prior2 = (prior_w + b2[0]).reshape(1, E).astype(jnp.float32)
    b1r = b1.reshape(1, hid)
    w2r = W2.reshape(1, hid)
    b3r = b3.reshape(1, W3.shape[1])

    node, attn = pl.pallas_call(
        _gnn_block,
        grid=grid,
        in_specs=[
            pl.BlockSpec((bb, E, D), lambda i: (i, 0, 0)),
            pl.BlockSpec((1, E), lambda i: (0, 0)),
            pl.BlockSpec(W1.shape, lambda i: (0, 0)),
            pl.BlockSpec((1, hid), lambda i: (0, 0)),
            pl.BlockSpec((1, hid), lambda i: (0, 0)),
            pl.BlockSpec(W3.shape, lambda i: (0, 0)),
            pl.BlockSpec((1, W3.shape[1]), lambda i: (0, 0)),
        ],
        out_specs=[
            pl.BlockSpec((bb, NUM_NODES, W3.shape[1]), lambda i: (i, 0, 0)),
            pl.BlockSpec((bb, E), lambda i: (i, 0)),
        ],
        out_shape=[
            jax.ShapeDtypeStruct((B, NUM_NODES, W3.shape[1]), jnp.float32),
            jax.ShapeDtypeStruct((B, E), jnp.float32),
        ],
    )(edge_feats, prior2, W1, b1r, w2r, W3, b3r)
    return node, attn


# speedup vs baseline: 2.4602x; 2.4602x over previous
import jax
import jax.numpy as jnp
from jax.experimental import pallas as pl

NUM_EDGES = 36
NUM_NODES = 12


def _gnn_block(x_ref, prior_ref, w1_ref, b1_ref, w2_ref, w3_ref, b3_ref,
               node_ref, attn_ref):
    bb = x_ref.shape[1]
    d = x_ref.shape[2]
    x2 = x_ref[...].reshape(NUM_EDGES * bb, d)

    h = jnp.dot(x2.astype(jnp.bfloat16), w1_ref[...].astype(jnp.bfloat16),
                preferred_element_type=jnp.float32)
    h = jnp.maximum(h + b1_ref[...], 0.0)
    lcol = jnp.sum(h * w2_ref[...], axis=1, keepdims=True)

    logits = jnp.concatenate(
        [lcol[e * bb:(e + 1) * bb] for e in range(NUM_EDGES)], axis=1)
    logits = logits + prior_ref[...]
    m = jnp.max(logits, axis=1, keepdims=True)
    p = jnp.exp(logits - m)
    attn = p / jnp.sum(p, axis=1, keepdims=True)
    attn_ref[...] = attn

    acol = jnp.concatenate(
        [attn[:, e:e + 1] for e in range(NUM_EDGES)], axis=0)
    wt = x2 * acol

    nodes = []
    for u in range(6):
        acc = wt[(6 * u) * bb:(6 * u + 1) * bb]
        for k in range(1, 6):
            acc = acc + wt[(6 * u + k) * bb:(6 * u + k + 1) * bb]
        nodes.append(acc)
    for v in range(6):
        acc = wt[v * bb:(v + 1) * bb]
        for k in range(1, 6):
            acc = acc + wt[(6 * k + v) * bb:(6 * k + v + 1) * bb]
        nodes.append(acc)
    nstack = jnp.concatenate(nodes, axis=0)

    y = jnp.dot(nstack.astype(jnp.bfloat16), w3_ref[...].astype(jnp.bfloat16),
                preferred_element_type=jnp.float32)
    y = jnp.maximum(y + b3_ref[...], 0.0)
    node_ref[...] = y.reshape(NUM_NODES, bb, d)


def kernel(edge_feats, prior_w, W1, b1, W2, b2, W3, b3):
    B, E, D = edge_feats.shape
    hid = W1.shape[1]
    bb = 64
    while B % bb:
        bb //= 2
    grid = (B // bb,)

    x_t = jnp.transpose(edge_feats, (1, 0, 2))
    prior2 = (prior_w + b2[0]).reshape(1, E).astype(jnp.float32)
    b1r = b1.reshape(1, hid)
    w2r = W2.reshape(1, hid)
    b3r = b3.reshape(1, W3.shape[1])

    node_t, attn = pl.pallas_call(
        _gnn_block,
        grid=grid,
        in_specs=[
            pl.BlockSpec((E, bb, D), lambda i: (0, i, 0)),
            pl.BlockSpec((1, E), lambda i: (0, 0)),
            pl.BlockSpec(W1.shape, lambda i: (0, 0)),
            pl.BlockSpec((1, hid), lambda i: (0, 0)),
            pl.BlockSpec((1, hid), lambda i: (0, 0)),
            pl.BlockSpec(W3.shape, lambda i: (0, 0)),
            pl.BlockSpec((1, W3.shape[1]), lambda i: (0, 0)),
        ],
        out_specs=[
            pl.BlockSpec((NUM_NODES, bb, W3.shape[1]), lambda i: (0, i, 0)),
            pl.BlockSpec((bb, E), lambda i: (i, 0)),
        ],
        out_shape=[
            jax.ShapeDtypeStruct((NUM_NODES, B, W3.shape[1]), jnp.float32),
            jax.ShapeDtypeStruct((B, E), jnp.float32),
        ],
    )(x_t, prior2, W1, b1r, w2r, W3, b3r)
    node = jnp.transpose(node_t, (1, 0, 2))
    return node, attn


# speedup vs baseline: 2.8754x; 1.1687x over previous
import jax
import jax.numpy as jnp
from jax.experimental import pallas as pl

NUM_EDGES = 36
NUM_NODES = 12


def _gnn_block(x_ref, prior_ref, w1_ref, b1_ref, w2_ref, w3_ref, b3_ref,
               node_ref, attn_ref):
    bb = x_ref.shape[1]
    d = x_ref.shape[2]
    x2 = x_ref[...].reshape(NUM_EDGES * bb, d)

    h = jnp.dot(x2.astype(jnp.bfloat16), w1_ref[...].astype(jnp.bfloat16),
                preferred_element_type=jnp.float32)
    h = jnp.maximum(h + b1_ref[...], 0.0)
    lcol = jnp.sum(h * w2_ref[...], axis=1, keepdims=True)

    logits = jnp.concatenate(
        [lcol[e * bb:(e + 1) * bb] for e in range(NUM_EDGES)], axis=1)
    logits = logits + prior_ref[...]
    m = jnp.max(logits, axis=1, keepdims=True)
    p = jnp.exp(logits - m)
    attn = p / jnp.sum(p, axis=1, keepdims=True)
    attn_ref[...] = attn

    wt = [x2[e * bb:(e + 1) * bb] * attn[:, e:e + 1]
          for e in range(NUM_EDGES)]

    nodes = []
    for u in range(6):
        acc = wt[6 * u]
        for k in range(1, 6):
            acc = acc + wt[6 * u + k]
        nodes.append(acc)
    for v in range(6):
        acc = wt[v]
        for k in range(1, 6):
            acc = acc + wt[6 * k + v]
        nodes.append(acc)
    nstack = jnp.concatenate(nodes, axis=0)

    y = jnp.dot(nstack.astype(jnp.bfloat16), w3_ref[...].astype(jnp.bfloat16),
                preferred_element_type=jnp.float32)
    y = jnp.maximum(y + b3_ref[...], 0.0)
    node_ref[...] = y.reshape(NUM_NODES, bb, d)


def kernel(edge_feats, prior_w, W1, b1, W2, b2, W3, b3):
    B, E, D = edge_feats.shape
    hid = W1.shape[1]
    bb = 128
    while B % bb:
        bb //= 2
    grid = (B // bb,)

    x_t = jnp.transpose(edge_feats, (1, 0, 2))
    prior2 = (prior_w + b2[0]).reshape(1, E).astype(jnp.float32)
    b1r = b1.reshape(1, hid)
    w2r = W2.reshape(1, hid)
    b3r = b3.reshape(1, W3.shape[1])

    node_t, attn = pl.pallas_call(
        _gnn_block,
        grid=grid,
        in_specs=[
            pl.BlockSpec((E, bb, D), lambda i: (0, i, 0)),
            pl.BlockSpec((1, E), lambda i: (0, 0)),
            pl.BlockSpec(W1.shape, lambda i: (0, 0)),
            pl.BlockSpec((1, hid), lambda i: (0, 0)),
            pl.BlockSpec((1, hid), lambda i: (0, 0)),
            pl.BlockSpec(W3.shape, lambda i: (0, 0)),
            pl.BlockSpec((1, W3.shape[1]), lambda i: (0, 0)),
        ],
        out_specs=[
            pl.BlockSpec((NUM_NODES, bb, W3.shape[1]), lambda i: (0, i, 0)),
            pl.BlockSpec((bb, E), lambda i: (i, 0)),
        ],
        out_shape=[
            jax.ShapeDtypeStruct((NUM_NODES, B, W3.shape[1]), jnp.float32),
            jax.ShapeDtypeStruct((B, E), jnp.float32),
        ],
    )(x_t, prior2, W1, b1r, w2r, W3, b3r)
    node = jnp.transpose(node_t, (1, 0, 2))
    return node, attn
